# baseline (speedup 1.0000x reference)
import functools
import math

import jax
import jax.numpy as jnp
import numpy as np
from jax import lax
from jax.experimental import pallas as pl
from jax.experimental.pallas import tpu as pltpu
from jax.experimental.pallas import tpu_sc as plsc


def _hilbert_curve(depth):
    curve = np.zeros(shape=(4 ** depth, 2)).astype(np.int32)
    curve[0:4, :] = [[0, 0], [0, 1], [1, 1], [1, 0]]
    step = 1
    size = 1
    for _ in range(2, depth + 1):
        step *= 2
        size *= 4
        fx = np.copy(curve[0:size, 0])
        fy = np.copy(curve[0:size, 1])
        curve[0:size, 0] = fy
        curve[0:size, 1] = fx
        curve[size:size * 2, 0] = fx
        curve[size:size * 2, 1] = fy + step
        curve[size * 2:size * 3, 0] = fx + step
        curve[size * 2:size * 3, 1] = fy + step
        curve[size * 3:size * 4, 0] = step * 2 - 1 - fy
        curve[size * 3:size * 4, 1] = step - 1 - fx
    return curve


def _axis_coords(extent, depth):
    step_size = extent / 2 ** depth
    ceil = np.ceil(step_size)
    floor = np.floor(step_size)
    if np.abs(step_size - int(step_size)) > 0.001:
        if np.abs(step_size - int(step_size) - 0.5) < 0.001:
            def add_fn(i):
                return [ceil, floor][i % 2]
        elif np.abs(step_size - int(step_size)) > 0.7:
            def add_fn(i):
                return [ceil, ceil, ceil, floor][i % 4]
        else:
            def add_fn(i):
                return ceil
    else:
        def add_fn(i):
            return ceil
    begin = max(floor - np.ceil(ceil / 2), 0)
    coords = []
    i = 0
    while begin < extent:
        coords.append(int(begin))
        begin += add_fn(i)
        i += 1
    return coords


_B, _H, _W, _C = 4, 224, 224, 96
_DEPTH = 7

_lg = math.log(_H, 2)
_closest = min((math.floor(_lg), math.ceil(_lg)), key=lambda z: abs(_H - 2 ** z))
_MAXD = min(_closest if 2 ** _closest <= _H else _closest - 1, _DEPTH)
_CURVE = _hilbert_curve(_MAXD)
_CX = np.asarray(_axis_coords(_H, _MAXD), np.int32)
_CY = np.asarray(_axis_coords(_W, _MAXD), np.int32)
_N = _CURVE.shape[0]
_NSIDE = 2 ** _MAXD

_NW = 32
_NSLAB = _B * _NSIDE
_SLABS_PER_W = _NSLAB // _NW
_PTS = _N // _NSIDE

_order = np.argsort(_CURVE[:, 0], kind="stable")
_pos_by_q = _order.reshape(_NSIDE, _PTS)
_hy_by_q = _CY[_CURVE[_pos_by_q, 1]]

_SID = np.zeros((_NSLAB,), np.int32)
_WIDX = np.zeros((_NSLAB, _PTS), np.int32)
_NIDX = np.zeros((_NSLAB, _PTS), np.int32)
def _bank_order(wrow):
    buckets = [[] for _ in range(16)]
    for p in range(wrow.shape[0]):
        buckets[wrow[p] % 16].append(p)
    order = []
    for g in range(8):
        for r in range(16):
            order.append(buckets[r][g])
    return np.asarray(order, np.int64)


for _b in range(_B):
    for _q in range(_NSIDE):
        _s = _b * _NSIDE + _q
        _SID[_s] = _b * _H + _CX[_q]
        _perm = _bank_order(_hy_by_q[_q])
        _WIDX[_s] = _hy_by_q[_q][_perm]
        _NIDX[_s] = _b * _N + _pos_by_q[_q][_perm]

_SID_T = np.zeros((_NW, 128), np.int32)
_SID_T[:, :_SLABS_PER_W] = _SID.reshape(_NW, _SLABS_PER_W)
_WIDX_T = _WIDX.reshape(_NW, _SLABS_PER_W, _PTS)
_NIDX_T = _NIDX.reshape(_NW, _SLABS_PER_W, _PTS)


_ROWS = _B * _N
_CB = _C // 16


@functools.lru_cache(maxsize=1)
def _build():
    mesh = plsc.VectorSubcoreMesh(core_axis_name="c", subcore_axis_name="s")

    @functools.partial(
        pl.kernel,
        mesh=mesh,
        out_type=jax.ShapeDtypeStruct((_ROWS, 128), jnp.float32),
        scratch_types=[
            pltpu.VMEM((128,), jnp.int32),
            pltpu.VMEM((_SLABS_PER_W, _PTS), jnp.int32),
            pltpu.VMEM((_SLABS_PER_W, _PTS), jnp.int32),
            [pltpu.VMEM((_C, _W), jnp.float32) for _ in range(2)],
            [pltpu.VMEM((_PTS, 128), jnp.float32) for _ in range(2)],
            [pltpu.SemaphoreType.DMA for _ in range(2)],
            [pltpu.SemaphoreType.DMA for _ in range(2)],
            pltpu.SemaphoreType.DMA,
        ],
        compiler_params=pltpu.CompilerParams(use_tc_tiling_on_sc=True,
                                             needs_layout_passes=False),
    )
    def hilbert_gather(x3_hbm, sid_hbm, widx_hbm, nidx_hbm, out_hbm,
                       sid_v, widx_v, nidx_v, slab, stage, gsem, ssem, isem):
        wid = lax.axis_index("s") * 2 + lax.axis_index("c")
        pltpu.async_copy(sid_hbm.at[wid], sid_v, isem).wait()
        pltpu.async_copy(widx_hbm.at[wid], widx_v, isem).wait()
        pltpu.async_copy(nidx_hbm.at[wid], nidx_v, isem).wait()
        sids = [sid_v[pl.ds(0, 16)][j] for j in range(_SLABS_PER_W)]

        _LAG = 4

        def assemble(j, slab_ref, stage_ref):
            def blk(k, carry):
                wv = widx_v[j, pl.ds(k * 16, 16)]
                rows = lax.iota(jnp.int32, 16) + k * 16
                vals = [None] * _C
                cvecs = [None] * _C
                lanes = lax.iota(jnp.int32, 16)
                for c in range(_C + _LAG):
                    if c < _C:
                        t = lanes + c
                        if c > _C - 16:
                            t = t - jnp.where(t >= _C, _C, 0)
                        cvecs[c] = t
                        vals[c] = plsc.load_gather(slab_ref, [cvecs[c], wv])
                    if c >= _LAG:
                        plsc.store_scatter(stage_ref, [rows, cvecs[c - _LAG]],
                                           vals[c - _LAG])
                return carry

            lax.fori_loop(0, _PTS // 16, blk, 0, unroll=False)

        gather_h = [None] * _SLABS_PER_W
        scatter_h = [None] * _SLABS_PER_W
        gather_h[0] = pltpu.async_copy(x3_hbm.at[sids[0]], slab[0], gsem[0])
        for j in range(_SLABS_PER_W):
            b = j % 2
            if j + 1 < _SLABS_PER_W:
                gather_h[j + 1] = pltpu.async_copy(
                    x3_hbm.at[sids[j + 1]], slab[(j + 1) % 2], gsem[(j + 1) % 2])
            gather_h[j].wait()
            if j >= 2:
                scatter_h[j - 2].wait()
            assemble(j, slab[b], stage[b])
            scatter_h[j] = pltpu.async_copy(
                stage[b], out_hbm.at[nidx_v.at[j]], ssem[b])
        scatter_h[_SLABS_PER_W - 2].wait()
        scatter_h[_SLABS_PER_W - 1].wait()

    return hilbert_gather


def _tc_slice_body(x_ref, o_ref):
    o_ref[...] = x_ref[:, :_C]


@functools.lru_cache(maxsize=1)
def _build_tc_slice():
    return pl.pallas_call(
        _tc_slice_body,
        out_shape=jax.ShapeDtypeStruct((_ROWS, _C), jnp.float32),
        grid=(32,),
        in_specs=[pl.BlockSpec((_ROWS // 32, 128), lambda i: (i, 0))],
        out_specs=pl.BlockSpec((_ROWS // 32, _C), lambda i: (i, 0)),
    )


def kernel(x):
    xt = jnp.transpose(x, (0, 1, 3, 2)).reshape(_B * _H, _C, _W)
    outp = _build()(xt, jnp.asarray(_SID_T), jnp.asarray(_WIDX_T),
                    jnp.asarray(_NIDX_T))
    return _build_tc_slice()(outp).reshape(_B, _N, _C)


# speedup vs baseline: 1.3453x; 1.3453x over previous
import functools
import math

import jax
import jax.numpy as jnp
import numpy as np
from jax import lax
from jax.experimental import pallas as pl
from jax.experimental.pallas import tpu as pltpu
from jax.experimental.pallas import tpu_sc as plsc


def _hilbert_curve(depth):
    curve = np.zeros(shape=(4 ** depth, 2)).astype(np.int32)
    curve[0:4, :] = [[0, 0], [0, 1], [1, 1], [1, 0]]
    step = 1
    size = 1
    for _ in range(2, depth + 1):
        step *= 2
        size *= 4
        fx = np.copy(curve[0:size, 0])
        fy = np.copy(curve[0:size, 1])
        curve[0:size, 0] = fy
        curve[0:size, 1] = fx
        curve[size:size * 2, 0] = fx
        curve[size:size * 2, 1] = fy + step
        curve[size * 2:size * 3, 0] = fx + step
        curve[size * 2:size * 3, 1] = fy + step
        curve[size * 3:size * 4, 0] = step * 2 - 1 - fy
        curve[size * 3:size * 4, 1] = step - 1 - fx
    return curve


def _axis_coords(extent, depth):
    step_size = extent / 2 ** depth
    ceil = np.ceil(step_size)
    floor = np.floor(step_size)
    if np.abs(step_size - int(step_size)) > 0.001:
        if np.abs(step_size - int(step_size) - 0.5) < 0.001:
            def add_fn(i):
                return [ceil, floor][i % 2]
        elif np.abs(step_size - int(step_size)) > 0.7:
            def add_fn(i):
                return [ceil, ceil, ceil, floor][i % 4]
        else:
            def add_fn(i):
                return ceil
    else:
        def add_fn(i):
            return ceil
    begin = max(floor - np.ceil(ceil / 2), 0)
    coords = []
    i = 0
    while begin < extent:
        coords.append(int(begin))
        begin += add_fn(i)
        i += 1
    return coords


_B, _H, _W, _C = 4, 224, 224, 96
_DEPTH = 7

_lg = math.log(_H, 2)
_closest = min((math.floor(_lg), math.ceil(_lg)), key=lambda z: abs(_H - 2 ** z))
_MAXD = min(_closest if 2 ** _closest <= _H else _closest - 1, _DEPTH)
_CURVE = _hilbert_curve(_MAXD)
_CX = np.asarray(_axis_coords(_H, _MAXD), np.int32)
_CY = np.asarray(_axis_coords(_W, _MAXD), np.int32)
_N = _CURVE.shape[0]
_NSIDE = 2 ** _MAXD

_NW = 32
_NSLAB = _B * _NSIDE
_SLABS_PER_W = _NSLAB // _NW
_PTS = _N // _NSIDE

_order = np.argsort(_CURVE[:, 0], kind="stable")
_pos_by_q = _order.reshape(_NSIDE, _PTS)
_hy_by_q = _CY[_CURVE[_pos_by_q, 1]]

_SID = np.zeros((_NSLAB,), np.int32)
_WIDX = np.zeros((_NSLAB, _PTS), np.int32)
_NIDX = np.zeros((_NSLAB, _PTS), np.int32)
def _bank_order(wrow):
    buckets = [[] for _ in range(16)]
    for p in range(wrow.shape[0]):
        buckets[wrow[p] % 16].append(p)
    order = []
    for g in range(8):
        for r in range(16):
            order.append(buckets[r][g])
    return np.asarray(order, np.int64)


for _b in range(_B):
    for _q in range(_NSIDE):
        _s = _b * _NSIDE + _q
        _SID[_s] = _b * _H + _CX[_q]
        _perm = _bank_order(_hy_by_q[_q])
        _WIDX[_s] = _hy_by_q[_q][_perm]
        _NIDX[_s] = _b * _N + _pos_by_q[_q][_perm]

_SID_T = np.zeros((_NW, 128), np.int32)
_SID_T[:, :_SLABS_PER_W] = _SID.reshape(_NW, _SLABS_PER_W)
_WIDX_T = _WIDX.reshape(_NW, _SLABS_PER_W, _PTS)
_NIDX_T = _NIDX.reshape(_NW, _SLABS_PER_W, _PTS)


_ROWS = _B * _N
_CB = _C // 16


@functools.lru_cache(maxsize=1)
def _build():
    mesh = plsc.VectorSubcoreMesh(core_axis_name="c", subcore_axis_name="s")

    @functools.partial(
        pl.kernel,
        mesh=mesh,
        out_type=jax.ShapeDtypeStruct((_ROWS, 128), jnp.float32),
        scratch_types=[
            pltpu.VMEM((128,), jnp.int32),
            pltpu.VMEM((_SLABS_PER_W, _PTS), jnp.int32),
            pltpu.VMEM((_SLABS_PER_W, _PTS), jnp.int32),
            [pltpu.VMEM((_C, _W), jnp.float32) for _ in range(2)],
            [pltpu.VMEM((_PTS, 128), jnp.float32) for _ in range(2)],
            [pltpu.SemaphoreType.DMA for _ in range(2)],
            [pltpu.SemaphoreType.DMA for _ in range(2)],
            pltpu.SemaphoreType.DMA,
        ],
        compiler_params=pltpu.CompilerParams(use_tc_tiling_on_sc=True,
                                             needs_layout_passes=False),
    )
    def hilbert_gather(x3_hbm, sid_hbm, widx_hbm, nidx_hbm, out_hbm,
                       sid_v, widx_v, nidx_v, slab, stage, gsem, ssem, isem):
        wid = lax.axis_index("s") * 2 + lax.axis_index("c")
        pltpu.async_copy(sid_hbm.at[wid], sid_v, isem).wait()
        pltpu.async_copy(widx_hbm.at[wid], widx_v, isem).wait()
        pltpu.async_copy(nidx_hbm.at[wid], nidx_v, isem).wait()
        sids = [sid_v[pl.ds(0, 16)][j] for j in range(_SLABS_PER_W)]

        _LAG = 4

        def assemble(j, slab_ref, stage_ref):
            def blk(k, carry):
                wv = widx_v[j, pl.ds(k * 16, 16)]
                rows = lax.iota(jnp.int32, 16) + k * 16
                vals = [None] * _C
                cvecs = [None] * _C
                lanes = lax.iota(jnp.int32, 16)
                for c in range(_C + _LAG):
                    if c < _C:
                        t = lanes + c
                        if c > _C - 16:
                            t = t - jnp.where(t >= _C, _C, 0)
                        cvecs[c] = t
                        vals[c] = plsc.load_gather(slab_ref, [cvecs[c], wv])
                    if c >= _LAG:
                        plsc.store_scatter(stage_ref, [rows, cvecs[c - _LAG]],
                                           vals[c - _LAG])
                return carry

            lax.fori_loop(0, _PTS // 16, blk, 0, unroll=False)

        gather_h = [None] * _SLABS_PER_W
        scatter_h = [None] * _SLABS_PER_W
        gather_h[0] = pltpu.async_copy(x3_hbm.at[sids[0]], slab[0], gsem[0])
        for j in range(_SLABS_PER_W):
            b = j % 2
            if j + 1 < _SLABS_PER_W:
                gather_h[j + 1] = pltpu.async_copy(
                    x3_hbm.at[sids[j + 1]], slab[(j + 1) % 2], gsem[(j + 1) % 2])
            gather_h[j].wait()
            if j >= 2:
                scatter_h[j - 2].wait()
            assemble(j, slab[b], stage[b])
            scatter_h[j] = pltpu.async_copy(
                stage[b], out_hbm.at[nidx_v.at[j]], ssem[b])
        scatter_h[_SLABS_PER_W - 2].wait()
        scatter_h[_SLABS_PER_W - 1].wait()

    return hilbert_gather


def kernel(x):
    xt = jnp.transpose(x, (0, 1, 3, 2)).reshape(_B * _H, _C, _W)
    outp = _build()(xt, jnp.asarray(_SID_T), jnp.asarray(_WIDX_T),
                    jnp.asarray(_NIDX_T))
    return outp[:, :_C].reshape(_B, _N, _C)
